# baseline (speedup 1.0000x reference)
import functools

import jax
import jax.numpy as jnp
from jax import lax
from jax.experimental import pallas as pl
from jax.experimental.pallas import tpu as pltpu
from jax.experimental.pallas import tpu_sc as plsc

_N = 1024
_ROWS = 18432
_CH = 16
_LANES = 16
_SLICES = _N // _LANES
_BIG = 1 << 30

_TC_ROWS = 3072


def _sc_argmax_body(x_hbm, idx_hbm, buf, obuf, sem):
    nc = plsc.get_sparse_core_info().num_cores
    wid = lax.axis_index("s") * nc + lax.axis_index("c")
    nworkers = 32
    rpw = _ROWS // nworkers
    nch = rpw // _CH
    base = wid * rpw

    pltpu.make_async_copy(x_hbm.at[pl.ds(base, _CH)], buf.at[0], sem).start()

    def chunk_body(k, carry):
        slot = lax.rem(k, 2)
        nslot = lax.rem(k + 1, 2)

        @pl.when(k + 1 < nch)
        def _():
            pltpu.make_async_copy(
                x_hbm.at[pl.ds(base + (k + 1) * _CH, _CH)], buf.at[nslot], sem
            ).start()

        pltpu.make_async_copy(
            x_hbm.at[pl.ds(base + k * _CH, _CH)], buf.at[slot], sem
        ).wait()

        riota = lax.iota(jnp.int32, _LANES)
        zero = jnp.zeros((_LANES,), jnp.int32)
        m = plsc.load_gather(buf.at[slot], [riota, zero])
        idxv = zero

        def p_body(p, mc):
            mm, ii = mc
            col = jnp.full((_LANES,), 0, jnp.int32) + p
            v = plsc.load_gather(buf.at[slot], [riota, col])
            gt = v > mm
            mm = jnp.where(gt, v, mm)
            ii = jnp.where(gt, col, ii)
            return (mm, ii)

        m, idxv = lax.fori_loop(1, _N, p_body, (m, idxv), unroll=8)
        obuf[slot, :] = idxv
        pltpu.sync_copy(obuf.at[slot], idx_hbm.at[pl.ds(base + k * _CH, _CH)])
        return carry

    lax.fori_loop(0, nch, chunk_body, 0, unroll=False)


@functools.partial(
    pl.kernel,
    mesh=plsc.VectorSubcoreMesh(core_axis_name="c", subcore_axis_name="s"),
    out_type=jax.ShapeDtypeStruct((_ROWS,), jnp.int32),
    scratch_types=[
        pltpu.VMEM((2, _CH, _N), jnp.float32),
        pltpu.VMEM((2, _CH), jnp.int32),
        pltpu.SemaphoreType.DMA,
    ],
    compiler_params=pltpu.CompilerParams(
        use_tc_tiling_on_sc=False, needs_layout_passes=False
    ),
)
def _sc_argmax(x_hbm, idx_hbm, buf, obuf, sem):
    _sc_argmax_body(x_hbm, idx_hbm, buf, obuf, sem)


def _onehot_write_block(idx_ref, o_ref):
    iota = lax.broadcasted_iota(jnp.int32, o_ref.shape, 1)
    o_ref[...] = (iota == idx_ref[...]).astype(o_ref.dtype)


def kernel(x):
    b, s, n = x.shape
    rows = b * s
    x2 = x.reshape(rows, n)
    idx = _sc_argmax(x2)
    idx2 = idx.reshape(rows, 1)
    out = pl.pallas_call(
        _onehot_write_block,
        grid=(rows // _TC_ROWS,),
        in_specs=[pl.BlockSpec((_TC_ROWS, 1), lambda i: (i, 0))],
        out_specs=pl.BlockSpec((_TC_ROWS, n), lambda i: (i, 0)),
        out_shape=jax.ShapeDtypeStruct((rows, n), x.dtype),
    )(idx2)
    return out.reshape(b, s, n)


# speedup vs baseline: 1.0861x; 1.0861x over previous
import functools

import jax
import jax.numpy as jnp
from jax import lax
from jax.experimental import pallas as pl
from jax.experimental.pallas import tpu as pltpu
from jax.experimental.pallas import tpu_sc as plsc

_N = 1024
_ROWS = 18432
_CH = 16
_LANES = 16
_SLICES = _N // _LANES
_BIG = 1 << 30

_TC_ROWS = 3072


def _sc_argmax_body(x_hbm, idx_hbm, buf, obuf, sem):
    nc = plsc.get_sparse_core_info().num_cores
    wid = lax.axis_index("s") * nc + lax.axis_index("c")
    nworkers = 32
    rpw = _ROWS // nworkers
    nch = rpw // _CH
    base = wid * rpw

    pltpu.make_async_copy(x_hbm.at[pl.ds(base, _CH)], buf.at[0], sem).start()

    def chunk_body(k, carry):
        slot = lax.rem(k, 2)
        nslot = lax.rem(k + 1, 2)

        @pl.when(k + 1 < nch)
        def _():
            pltpu.make_async_copy(
                x_hbm.at[pl.ds(base + (k + 1) * _CH, _CH)], buf.at[nslot], sem
            ).start()

        pltpu.make_async_copy(
            x_hbm.at[pl.ds(base + k * _CH, _CH)], buf.at[slot], sem
        ).wait()

        riota = lax.iota(jnp.int32, _LANES)
        zero = jnp.zeros((_LANES,), jnp.int32)
        m = plsc.load_gather(buf.at[slot], [riota, zero])
        idxv = zero
        unroll = 8

        def p_body(g, mc):
            mm, ii = mc
            p0 = g * unroll
            cols = [jnp.full((_LANES,), 0, jnp.int32) + (p0 + u) for u in range(unroll)]
            vals = [plsc.load_gather(buf.at[slot], [riota, c]) for c in cols]
            while len(vals) > 1:
                nv, nc = [], []
                for a in range(0, len(vals), 2):
                    gt = vals[a + 1] > vals[a]
                    nv.append(jnp.where(gt, vals[a + 1], vals[a]))
                    nc.append(jnp.where(gt, cols[a + 1], cols[a]))
                vals, cols = nv, nc
            gt = vals[0] > mm
            mm = jnp.where(gt, vals[0], mm)
            ii = jnp.where(gt, cols[0], ii)
            return (mm, ii)

        m, idxv = lax.fori_loop(0, _N // unroll, p_body, (m, idxv), unroll=2)
        obuf[slot, :] = idxv
        pltpu.sync_copy(obuf.at[slot], idx_hbm.at[pl.ds(base + k * _CH, _CH)])
        return carry

    lax.fori_loop(0, nch, chunk_body, 0, unroll=False)


@functools.partial(
    pl.kernel,
    mesh=plsc.VectorSubcoreMesh(core_axis_name="c", subcore_axis_name="s"),
    out_type=jax.ShapeDtypeStruct((_ROWS,), jnp.int32),
    scratch_types=[
        pltpu.VMEM((2, _CH, _N), jnp.float32),
        pltpu.VMEM((2, _CH), jnp.int32),
        pltpu.SemaphoreType.DMA,
    ],
    compiler_params=pltpu.CompilerParams(
        use_tc_tiling_on_sc=True, needs_layout_passes=False
    ),
)
def _sc_argmax(x_hbm, idx_hbm, buf, obuf, sem):
    _sc_argmax_body(x_hbm, idx_hbm, buf, obuf, sem)


def _onehot_write_block(idx_ref, o_ref):
    iota = lax.broadcasted_iota(jnp.int32, o_ref.shape, 1)
    o_ref[...] = (iota == idx_ref[...]).astype(o_ref.dtype)


def kernel(x):
    b, s, n = x.shape
    rows = b * s
    x2 = x.reshape(rows, n)
    idx = _sc_argmax(x2)
    idx2 = idx.reshape(rows, 1)
    out = pl.pallas_call(
        _onehot_write_block,
        grid=(rows // _TC_ROWS,),
        in_specs=[pl.BlockSpec((_TC_ROWS, 1), lambda i: (i, 0))],
        out_specs=pl.BlockSpec((_TC_ROWS, n), lambda i: (i, 0)),
        out_shape=jax.ShapeDtypeStruct((rows, n), x.dtype),
    )(idx2)
    return out.reshape(b, s, n)


# speedup vs baseline: 2.8509x; 2.6249x over previous
import functools

import jax
import jax.numpy as jnp
from jax import lax
from jax.experimental import pallas as pl
from jax.experimental.pallas import tpu as pltpu
from jax.experimental.pallas import tpu_sc as plsc

_N = 1024
_ROWS = 18432
_CH = 16
_LANES = 16
_SLICES = _N // _LANES
_BIG = 1 << 30

_TC_ROWS = 3072


def _sc_argmax_body(x_hbm, idx_hbm, buf, obuf, sem):
    nc = plsc.get_sparse_core_info().num_cores
    wid = lax.axis_index("s") * nc + lax.axis_index("c")
    nworkers = 32
    rpw = _ROWS // nworkers
    nch = rpw // _CH
    base = wid * rpw

    pltpu.make_async_copy(x_hbm.at[pl.ds(base, _CH)], buf.at[0], sem).start()

    def chunk_body(k, carry):
        slot = lax.rem(k, 2)
        nslot = lax.rem(k + 1, 2)

        @pl.when(k + 1 < nch)
        def _():
            pltpu.make_async_copy(
                x_hbm.at[pl.ds(base + (k + 1) * _CH, _CH)], buf.at[nslot], sem
            ).start()

        pltpu.make_async_copy(
            x_hbm.at[pl.ds(base + k * _CH, _CH)], buf.at[slot], sem
        ).wait()

        riota = lax.iota(jnp.int32, _LANES)
        zero = jnp.zeros((_LANES,), jnp.int32)

        def one_row(r):
            m = buf[slot, r, pl.ds(0, _LANES)]
            jv = zero
            for j in range(1, _SLICES):
                v = buf[slot, r, pl.ds(j * _LANES, _LANES)]
                gt = v > m
                m = jnp.where(gt, v, m)
                jv = jnp.where(gt, jnp.full((_LANES,), j, jnp.int32), jv)
            mx = jnp.max(m)
            lin = jv * _LANES + riota
            cand = jnp.where(m == mx, lin, _BIG)
            return jnp.min(cand)

        def row_body(h, acc):
            r0 = h * 2
            i0 = one_row(r0)
            i1 = one_row(r0 + 1)
            acc = jnp.where(riota == r0, jnp.full((_LANES,), 0, jnp.int32) + i0, acc)
            acc = jnp.where(riota == r0 + 1, jnp.full((_LANES,), 0, jnp.int32) + i1, acc)
            return acc

        idxv = lax.fori_loop(0, _CH // 2, row_body, zero)
        obuf[slot, :] = idxv
        pltpu.sync_copy(obuf.at[slot], idx_hbm.at[pl.ds(base + k * _CH, _CH)])
        return carry

    lax.fori_loop(0, nch, chunk_body, 0, unroll=False)


@functools.partial(
    pl.kernel,
    mesh=plsc.VectorSubcoreMesh(core_axis_name="c", subcore_axis_name="s"),
    out_type=jax.ShapeDtypeStruct((_ROWS,), jnp.int32),
    scratch_types=[
        pltpu.VMEM((2, _CH, _N), jnp.float32),
        pltpu.VMEM((2, _CH), jnp.int32),
        pltpu.SemaphoreType.DMA,
    ],
    compiler_params=pltpu.CompilerParams(
        use_tc_tiling_on_sc=True, needs_layout_passes=False
    ),
)
def _sc_argmax(x_hbm, idx_hbm, buf, obuf, sem):
    _sc_argmax_body(x_hbm, idx_hbm, buf, obuf, sem)


def _onehot_write_block(idx_ref, o_ref):
    iota = lax.broadcasted_iota(jnp.int32, o_ref.shape, 1)
    o_ref[...] = (iota == idx_ref[...]).astype(o_ref.dtype)


def kernel(x):
    b, s, n = x.shape
    rows = b * s
    x2 = x.reshape(rows, n)
    idx = _sc_argmax(x2)
    idx2 = idx.reshape(rows, 1)
    out = pl.pallas_call(
        _onehot_write_block,
        grid=(rows // _TC_ROWS,),
        in_specs=[pl.BlockSpec((_TC_ROWS, 1), lambda i: (i, 0))],
        out_specs=pl.BlockSpec((_TC_ROWS, n), lambda i: (i, 0)),
        out_shape=jax.ShapeDtypeStruct((rows, n), x.dtype),
    )(idx2)
    return out.reshape(b, s, n)


# speedup vs baseline: 4.9110x; 1.7227x over previous
import functools

import jax
import jax.numpy as jnp
from jax import lax
from jax.experimental import pallas as pl
from jax.experimental.pallas import tpu as pltpu
from jax.experimental.pallas import tpu_sc as plsc

_N = 1024
_ROWS = 18432
_LANES = 16
_SLICES = _N // _LANES
_BIG = 1 << 30
_CH = 16

_R = 3072
_NSC_BLOCKS = 2
_TC_BLOCKS = _ROWS // _R - _NSC_BLOCKS
_SC_ROWS = _NSC_BLOCKS * _R
_SC_BASE = _TC_BLOCKS * _R
_NWORKERS = 32


def _sc_argmax_body(x_hbm, idx_hbm, buf, obuf, sem):
    nc = plsc.get_sparse_core_info().num_cores
    wid = lax.axis_index("s") * nc + lax.axis_index("c")
    rpw = _SC_ROWS // _NWORKERS
    nch = rpw // _CH
    base = _SC_BASE + wid * rpw
    obase = wid * rpw

    pltpu.make_async_copy(x_hbm.at[pl.ds(base, _CH)], buf.at[0], sem).start()

    def chunk_body(k, carry):
        slot = lax.rem(k, 2)
        nslot = lax.rem(k + 1, 2)

        @pl.when(k + 1 < nch)
        def _():
            pltpu.make_async_copy(
                x_hbm.at[pl.ds(base + (k + 1) * _CH, _CH)], buf.at[nslot], sem
            ).start()

        pltpu.make_async_copy(
            x_hbm.at[pl.ds(base + k * _CH, _CH)], buf.at[slot], sem
        ).wait()

        riota = lax.iota(jnp.int32, _LANES)
        zero = jnp.zeros((_LANES,), jnp.int32)

        def one_row(r):
            m = buf[slot, r, pl.ds(0, _LANES)]
            jv = zero
            for j in range(1, _SLICES):
                v = buf[slot, r, pl.ds(j * _LANES, _LANES)]
                gt = v > m
                m = jnp.where(gt, v, m)
                jv = jnp.where(gt, jnp.full((_LANES,), j, jnp.int32), jv)
            mx = jnp.max(m)
            lin = jv * _LANES + riota
            cand = jnp.where(m == mx, lin, _BIG)
            return jnp.min(cand)

        def row_body(h, acc):
            r0 = h * 2
            i0 = one_row(r0)
            i1 = one_row(r0 + 1)
            acc = jnp.where(riota == r0, jnp.full((_LANES,), 0, jnp.int32) + i0, acc)
            acc = jnp.where(riota == r0 + 1, jnp.full((_LANES,), 0, jnp.int32) + i1, acc)
            return acc

        idxv = lax.fori_loop(0, _CH // 2, row_body, zero)
        plsc.store_scatter(obuf.at[slot], [riota, zero], idxv)
        pltpu.sync_copy(
            obuf.at[slot], idx_hbm.at[pl.ds(obase + k * _CH, _CH)]
        )
        return carry

    lax.fori_loop(0, nch, chunk_body, 0, unroll=False)


@functools.partial(
    pl.kernel,
    mesh=plsc.VectorSubcoreMesh(core_axis_name="c", subcore_axis_name="s"),
    out_type=jax.ShapeDtypeStruct((_SC_ROWS, 128), jnp.int32),
    scratch_types=[
        pltpu.VMEM((2, _CH, _N), jnp.float32),
        pltpu.VMEM((2, _CH, 128), jnp.int32),
        pltpu.SemaphoreType.DMA,
    ],
    compiler_params=pltpu.CompilerParams(
        use_tc_tiling_on_sc=True, needs_layout_passes=False
    ),
)
def _sc_argmax(x_hbm, idx_hbm, buf, obuf, sem):
    _sc_argmax_body(x_hbm, idx_hbm, buf, obuf, sem)


def _fused_onehot_block(x_ref, o_ref):
    x = x_ref[...]
    n = x.shape[1]
    m = jnp.max(x, axis=1, keepdims=True)
    iota = lax.broadcasted_iota(jnp.int32, x.shape, 1)
    idx = jnp.min(jnp.where(x == m, iota, n), axis=1, keepdims=True)
    o_ref[...] = (iota == idx).astype(o_ref.dtype)


def _onehot_from_idx_block(idx_ref, buf_ref, o_ref):
    del buf_ref
    idx = idx_ref[...][:, 0:1]
    iota = lax.broadcasted_iota(jnp.int32, o_ref.shape, 1)
    o_ref[...] = (iota == idx).astype(o_ref.dtype)


def kernel(x):
    b, s, n = x.shape
    rows = b * s
    x2 = x.reshape(rows, n)

    idxp = _sc_argmax(x2)

    buf = pl.pallas_call(
        _fused_onehot_block,
        grid=(_TC_BLOCKS,),
        in_specs=[pl.BlockSpec((_R, n), lambda i: (i, 0))],
        out_specs=pl.BlockSpec((_R, n), lambda i: (i, 0)),
        out_shape=jax.ShapeDtypeStruct((rows, n), x.dtype),
    )(x2)

    out = pl.pallas_call(
        _onehot_from_idx_block,
        grid=(_NSC_BLOCKS,),
        in_specs=[
            pl.BlockSpec((_R, 128), lambda i: (i, 0)),
            pl.BlockSpec(memory_space=pl.ANY),
        ],
        out_specs=pl.BlockSpec((_R, n), lambda i: (i + _TC_BLOCKS, 0)),
        out_shape=jax.ShapeDtypeStruct((rows, n), x.dtype),
        input_output_aliases={1: 0},
    )(idxp, buf)

    return out.reshape(b, s, n)


# speedup vs baseline: 7.1218x; 1.4502x over previous
import jax
import jax.numpy as jnp
from jax import lax
from jax.experimental import pallas as pl

_ROWS = 3072


def _onehot_argmax_block(x_ref, o_ref):
    x = x_ref[...]
    idx = jnp.argmax(x, axis=1)[:, None]
    iota = lax.broadcasted_iota(jnp.int32, x.shape, 1)
    o_ref[...] = (iota == idx).astype(o_ref.dtype)


def kernel(x):
    b, s, n = x.shape
    rows = b * s
    x2 = x.reshape(rows, n)
    out = pl.pallas_call(
        _onehot_argmax_block,
        grid=(rows // _ROWS,),
        in_specs=[pl.BlockSpec((_ROWS, n), lambda i: (i, 0))],
        out_specs=pl.BlockSpec((_ROWS, n), lambda i: (i, 0)),
        out_shape=jax.ShapeDtypeStruct((rows, n), x.dtype),
    )(x2)
    return out.reshape(b, s, n)
